# baseline (speedup 1.0000x reference)
import functools

import numpy as np
import jax
import jax.numpy as jnp
from jax import lax
from jax.experimental import pallas as pl
from jax.experimental.pallas import tpu as pltpu
from jax.experimental.pallas import tpu_sc as plsc

_B = 20
_NPER = 500
_NPAD = 512
_EPER = 8000
_D = 128
_N = _B * _NPER
_NW = 32
_GROWS = 384
_GPAD = _NW * _GROWS
_K1, _K2, _K3 = 450, 405, 365

_f32 = jnp.float32
_i32 = jnp.int32
_SIGN = np.int32(-(2 ** 31))


def _sc_mesh():
    return plsc.VectorSubcoreMesh(core_axis_name="c", subcore_axis_name="s")


_GB = _B * _NPAD
_GW = _GB // _NW
_GC = 64


def _sc_fused(item_idx, cat_idx, item_table, cat_table, src, dst, zeros):
    @functools.partial(
        pl.kernel,
        mesh=_sc_mesh(),
        out_type=[jax.ShapeDtypeStruct((_GB, _D), _f32),
                  jax.ShapeDtypeStruct((_GB, _D), _f32),
                  jax.ShapeDtypeStruct((_B, 128 * _NPAD * 4), _f32)],
        scratch_types=[pltpu.VMEM((_GW,), _i32),
                       pltpu.VMEM((_GW,), _i32),
                       pltpu.VMEM((_GC, _D), _f32),
                       pltpu.VMEM((_GC, _D), _f32),
                       pltpu.VMEM((_GC, _D), _f32),
                       pltpu.VMEM((_GC, _D), _f32),
                       pltpu.VMEM((128 * _NPAD,), _f32),
                       pltpu.VMEM((_EPER,), _i32),
                       pltpu.VMEM((_EPER,), _i32),
                       pltpu.SemaphoreType.DMA,
                       pltpu.SemaphoreType.DMA,
                       pltpu.SemaphoreType.DMA,
                       pltpu.SemaphoreType.DMA,
                       pltpu.SemaphoreType.DMA,
                       pltpu.SemaphoreType.DMA,
                       pltpu.SemaphoreType.DMA,
                       pltpu.SemaphoreType.DMA],
        compiler_params=pltpu.CompilerParams(needs_layout_passes=False),
    )
    def fused_k(item_idx_h, cat_idx_h, item_t, cat_t, src_h, dst_h, zeros_h,
                item_o, cat_o, a_h,
                iidx, cidx, ib0, ib1, cb0, cb1, abuf, sbuf, dbuf,
                gi0, gi1, gc0, gc1, so0, so1, so2, so3):
        wid = lax.axis_index("s") * 2 + lax.axis_index("c")
        base = pl.multiple_of(wid * _GW, 64)
        pltpu.sync_copy(item_idx_h.at[pl.ds(base, _GW)], iidx)
        pltpu.sync_copy(cat_idx_h.at[pl.ds(base, _GW)], cidx)
        ibufs, cbufs = (ib0, ib1), (cb0, cb1)
        gis, gcs = (gi0, gi1), (gc0, gc1)
        sis, scs = (so0, so1), (so2, so3)
        ones = jnp.full((16,), 1.0, _f32)

        def fire_item(k):
            return pltpu.async_copy(
                item_t.at[iidx.at[pl.ds(k * _GC, _GC)]],
                ibufs[k % 2], gis[k % 2])

        def fire_cat(k):
            return pltpu.async_copy(
                cat_t.at[cidx.at[pl.ds(k * _GC, _GC)]],
                cbufs[k % 2], gcs[k % 2])

        def store_item(k):
            return pltpu.async_copy(
                ibufs[k % 2], item_o.at[pl.ds(base + k * _GC, _GC)],
                sis[k % 2])

        def store_cat(k):
            return pltpu.async_copy(
                cbufs[k % 2], cat_o.at[pl.ds(base + k * _GC, _GC)],
                scs[k % 2])

        def a_task(t):
            task = t * _NW + wid

            @pl.when(task < _B * 4)
            def _():
                g = task // 4
                ch = task - g * 4
                row0 = ch * 128
                node0 = g * _NPER
                e0 = pl.multiple_of(g * _EPER, 8)
                pltpu.sync_copy(zeros_h, abuf)
                pltpu.sync_copy(src_h.at[pl.ds(e0, _EPER)], sbuf)
                pltpu.sync_copy(dst_h.at[pl.ds(e0, _EPER)], dbuf)

                def body(e, carry):
                    for u in range(4):
                        off = pl.multiple_of(e * 64 + u * 16, 8)
                        s = sbuf[pl.ds(off, 16)] - node0
                        d = dbuf[pl.ds(off, 16)] - (node0 + row0)
                        m = (d >= 0) & (d < 128)
                        flat = jnp.where(m, d, 0) * _NPAD + s
                        plsc.addupdate_scatter(abuf, [flat], ones, mask=m)
                    return carry

                lax.fori_loop(0, _EPER // 64, body, 0)
                dst0 = pl.multiple_of(ch * (128 * _NPAD), 8)
                pltpu.sync_copy(abuf, a_h.at[g, pl.ds(dst0, 128 * _NPAD)])

        g_i0, g_c0 = fire_item(0), fire_cat(0)
        g_i1, g_c1 = fire_item(1), fire_cat(1)
        a_task(0)
        g_i0.wait()
        st_i0 = store_item(0)
        g_c0.wait()
        st_c0 = store_cat(0)
        a_task(1)
        g_i1.wait()
        st_i1 = store_item(1)
        g_c1.wait()
        st_c1 = store_cat(1)
        st_i0.wait()
        g_i2 = fire_item(2)
        st_c0.wait()
        g_c2 = fire_cat(2)
        a_task(2)
        g_i2.wait()
        st_i2 = store_item(2)
        g_c2.wait()
        st_c2 = store_cat(2)
        st_i1.wait()
        g_i3 = fire_item(3)
        st_c1.wait()
        g_c3 = fire_cat(3)
        g_i3.wait()
        st_i3 = store_item(3)
        g_c3.wait()
        st_c3 = store_cat(3)
        st_i2.wait()
        g_i4 = fire_item(4)
        st_c2.wait()
        g_c4 = fire_cat(4)
        g_i4.wait()
        st_i4 = store_item(4)
        g_c4.wait()
        st_c4 = store_cat(4)
        st_i3.wait()
        st_c3.wait()
        st_i4.wait()
        st_c4.wait()

    return fused_k(item_idx, cat_idx, item_table, cat_table, src, dst, zeros)


_bf16 = jnp.bfloat16


def _int_matmul_split(a16, h):
    h_hi = h.astype(_bf16)
    h_lo = (h - h_hi.astype(_f32)).astype(_bf16)
    return (jnp.dot(a16, h_hi, preferred_element_type=_f32)
            + jnp.dot(a16, h_lo, preferred_element_type=_f32))


def _conv1_call(A, hi, hc, W_rel, W_root, b, p):
    def body(a_ref, hi_ref, hc_ref, wr_ref, wo_ref, b_ref, p_ref,
             h1_ref, sc_ref):
        a = a_ref[0]
        h0b = jnp.concatenate([hi_ref[0], hc_ref[0]], axis=1)
        hr = jnp.dot(h0b, wr_ref[...], preferred_element_type=_f32)
        h1 = _int_matmul_split(a, hr)
        h1 = h1 + jnp.dot(h0b, wo_ref[...], preferred_element_type=_f32)
        h1 = jnp.maximum(h1 + b_ref[...], 0.0)
        pv = p_ref[...]
        pn = jnp.sqrt(jnp.sum(pv * pv))
        raw = jnp.sum(h1 * pv, axis=1, keepdims=True) / pn
        valid = lax.broadcasted_iota(_i32, (_NPAD, 1), 0) < _NPER
        sc_ref[0] = jnp.where(valid, raw, -jnp.inf)
        h1_ref[0] = h1

    return pl.pallas_call(
        body,
        grid=(_B,),
        in_specs=[
            pl.BlockSpec((1, _NPAD, _NPAD), lambda g: (g, 0, 0)),
            pl.BlockSpec((1, _NPAD, _D), lambda g: (g, 0, 0)),
            pl.BlockSpec((1, _NPAD, _D), lambda g: (g, 0, 0)),
            pl.BlockSpec((2 * _D, _D), lambda g: (0, 0)),
            pl.BlockSpec((2 * _D, _D), lambda g: (0, 0)),
            pl.BlockSpec((1, _D), lambda g: (0, 0)),
            pl.BlockSpec((1, _D), lambda g: (0, 0)),
        ],
        out_specs=[
            pl.BlockSpec((1, _NPAD, _D), lambda g: (g, 0, 0)),
            pl.BlockSpec((1, _NPAD, 1), lambda g: (g, 0, 0)),
        ],
        out_shape=[jax.ShapeDtypeStruct((_B, _NPAD, _D), _f32),
                   jax.ShapeDtypeStruct((_B, _NPAD, 1), _f32)],
    )(A, hi, hc, W_rel, W_root, b, p)


def _conv_next(A, h, keep, gate, W_rel, W_root, b, p, k_prev):
    inv_k = 1.0 / float(k_prev)

    def body(a_ref, h_ref, k_ref, g_ref, wr_ref, wo_ref, b_ref, p_ref,
             x_ref, h2_ref, sc_ref):
        kp = k_ref[0]
        gt = g_ref[0]
        hp = h_ref[0] * gt * kp
        kept = kp > 0.0
        gmp = jnp.max(jnp.where(kept, hp, -jnp.inf), axis=0, keepdims=True)
        gap = jnp.sum(hp, axis=0, keepdims=True) * inv_k
        x_ref[0] = jnp.concatenate([gmp, gap], axis=1)
        a = a_ref[0]
        hr = jnp.dot(hp, wr_ref[...], preferred_element_type=_f32)
        h2 = _int_matmul_split(a, hr)
        h2 = h2 + jnp.dot(hp, wo_ref[...], preferred_element_type=_f32)
        h2 = jnp.maximum(h2 + b_ref[...], 0.0)
        pv = p_ref[...]
        pn = jnp.sqrt(jnp.sum(pv * pv))
        raw = jnp.sum(h2 * pv, axis=1, keepdims=True) / pn
        sc_ref[0] = jnp.where(kept, raw, -jnp.inf)
        h2_ref[0] = h2

    return pl.pallas_call(
        body,
        grid=(_B,),
        in_specs=[
            pl.BlockSpec((1, _NPAD, _NPAD), lambda g: (g, 0, 0)),
            pl.BlockSpec((1, _NPAD, _D), lambda g: (g, 0, 0)),
            pl.BlockSpec((1, _NPAD, 1), lambda g: (g, 0, 0)),
            pl.BlockSpec((1, _NPAD, 1), lambda g: (g, 0, 0)),
            pl.BlockSpec((_D, _D), lambda g: (0, 0)),
            pl.BlockSpec((_D, _D), lambda g: (0, 0)),
            pl.BlockSpec((1, _D), lambda g: (0, 0)),
            pl.BlockSpec((1, _D), lambda g: (0, 0)),
        ],
        out_specs=[
            pl.BlockSpec((1, 1, 2 * _D), lambda g: (g, 0, 0)),
            pl.BlockSpec((1, _NPAD, _D), lambda g: (g, 0, 0)),
            pl.BlockSpec((1, _NPAD, 1), lambda g: (g, 0, 0)),
        ],
        out_shape=[jax.ShapeDtypeStruct((_B, 1, 2 * _D), _f32),
                   jax.ShapeDtypeStruct((_B, _NPAD, _D), _f32),
                   jax.ShapeDtypeStruct((_B, _NPAD, 1), _f32)],
    )(A, h, keep, gate, W_rel, W_root, b, p)


def _pool_select(score2d, k):
    def body(s_ref, keep_ref, gate_ref):
        s = s_ref[...]
        bits = lax.bitcast_convert_type(s, _i32)
        key = jnp.where(bits >= 0, bits, bits ^ np.int32(0x7FFFFFFF))
        pref = jnp.zeros((_B, 1), _i32)
        for bit in range(31, -1, -1):
            m = _SIGN if bit == 31 else np.int32(1 << bit)
            cand = pref | m
            t = cand ^ _SIGN
            c = jnp.sum((key >= t).astype(_i32), axis=1, keepdims=True)
            pref = jnp.where(c >= k, cand, pref)
        kth = pref ^ _SIGN
        gtm = key > kth
        eq = (key >= kth) & (~gtm)
        n_gt = jnp.sum(gtm.astype(_i32), axis=1, keepdims=True)
        run = eq.astype(_i32)
        for sh in (1, 2, 4, 8, 16, 32, 64, 128, 256):
            shifted = jnp.concatenate(
                [jnp.zeros((_B, sh), _i32), run[:, : _NPAD - sh]], axis=1)
            run = run + shifted
        keep = gtm | (eq & (run <= (k - n_gt)))
        keep_ref[...] = keep.astype(_f32)
        gate_ref[...] = jnp.tanh(jnp.maximum(s, -100.0))

    return pl.pallas_call(
        body,
        out_shape=[jax.ShapeDtypeStruct((_B, _NPAD), _f32),
                   jax.ShapeDtypeStruct((_B, _NPAD), _f32)],
    )(score2d)


def _final_call(h3, keep, gate, x1, x2, emb_item, l1W, l1b, l2W, l2b):
    inv_k = 1.0 / float(_K3)

    def body(h_ref, k_ref, g_ref, x1_ref, x2_ref, e_ref,
             w1_ref, b1_ref, w2_ref, b2_ref, o_ref):
        kp = k_ref[0]
        gt = g_ref[0]
        hp = h_ref[0] * gt * kp
        kept = kp > 0.0
        gmp = jnp.max(jnp.where(kept, hp, -jnp.inf), axis=0, keepdims=True)
        gap = jnp.sum(hp, axis=0, keepdims=True) * inv_k
        gv = x1_ref[0] + x2_ref[0] + jnp.concatenate([gmp, gap], axis=1)
        gv = jnp.dot(gv, w1_ref[...], preferred_element_type=_f32)
        gv = jnp.maximum(gv + b1_ref[...], 0.0)
        gv = jnp.dot(gv, w2_ref[...], preferred_element_type=_f32)
        gv = jnp.maximum(gv + b2_ref[...], 0.0)
        logits = jnp.sum(e_ref[0] * gv, axis=1, keepdims=True)
        o_ref[0] = 1.0 / (1.0 + jnp.exp(-logits))

    return pl.pallas_call(
        body,
        grid=(_B,),
        in_specs=[
            pl.BlockSpec((1, _NPAD, _D), lambda g: (g, 0, 0)),
            pl.BlockSpec((1, _NPAD, 1), lambda g: (g, 0, 0)),
            pl.BlockSpec((1, _NPAD, 1), lambda g: (g, 0, 0)),
            pl.BlockSpec((1, 1, 2 * _D), lambda g: (g, 0, 0)),
            pl.BlockSpec((1, 1, 2 * _D), lambda g: (g, 0, 0)),
            pl.BlockSpec((1, _NPAD, _D), lambda g: (g, 0, 0)),
            pl.BlockSpec((2 * _D, 2 * _D), lambda g: (0, 0)),
            pl.BlockSpec((1, 2 * _D), lambda g: (0, 0)),
            pl.BlockSpec((2 * _D, _D), lambda g: (0, 0)),
            pl.BlockSpec((1, _D), lambda g: (0, 0)),
        ],
        out_specs=[pl.BlockSpec((1, _NPAD, 1), lambda g: (g, 0, 0))],
        out_shape=[jax.ShapeDtypeStruct((_B, _NPAD, 1), _f32)],
    )(h3, keep, gate, x1, x2, emb_item, l1W, l1b, l2W, l2b)


def kernel(x, edge_index, batch, item_table, cat_table, W1_rel, W1_root, b1,
           p1, W2_rel, W2_root, b2, p2, W3_rel, W3_root, b3, p3, lin1_W,
           lin1_b, lin2_W, lin2_b):
    idx_pad = jnp.zeros((_B, _NPAD - _NPER), _i32)
    item_idx = jnp.concatenate(
        [x[:, 0, 0].astype(_i32).reshape(_B, _NPER), idx_pad],
        axis=1).reshape(-1)
    cat_idx = jnp.concatenate(
        [x[:, 0, 1].astype(_i32).reshape(_B, _NPER), idx_pad],
        axis=1).reshape(-1)
    src = edge_index[0].astype(_i32)
    dst = edge_index[1].astype(_i32)
    emb_item_f, emb_cat_f, A_flat = _sc_fused(
        item_idx, cat_idx, item_table, cat_table, src, dst,
        jnp.zeros((128 * _NPAD,), _f32))
    emb_item = emb_item_f.reshape(_B, _NPAD, _D)
    emb_cat = emb_cat_f.reshape(_B, _NPAD, _D)
    A = A_flat.reshape(_B, _NPAD, _NPAD).astype(_bf16)

    h1, s1 = _conv1_call(A, emb_item, emb_cat, W1_rel, W1_root,
                         b1.reshape(1, _D), p1.reshape(1, _D))
    k1, g1 = _pool_select(s1.reshape(_B, _NPAD), _K1)
    keep1 = k1.reshape(_B, _NPAD, 1)
    gate1 = g1.reshape(_B, _NPAD, 1)

    x1, h2, s2 = _conv_next(A, h1, keep1, gate1, W2_rel, W2_root,
                            b2.reshape(1, _D), p2.reshape(1, _D), _K1)
    k2, g2 = _pool_select(s2.reshape(_B, _NPAD), _K2)
    keep2 = k2.reshape(_B, _NPAD, 1)
    gate2 = g2.reshape(_B, _NPAD, 1)

    x2, h3, s3 = _conv_next(A, h2, keep2, gate2, W3_rel, W3_root,
                            b3.reshape(1, _D), p3.reshape(1, _D), _K2)
    k3, g3 = _pool_select(s3.reshape(_B, _NPAD), _K3)
    keep3 = k3.reshape(_B, _NPAD, 1)
    gate3 = g3.reshape(_B, _NPAD, 1)

    (outp,) = _final_call(h3, keep3, gate3, x1, x2, emb_item,
                          lin1_W,
                          lin1_b.reshape(1, 2 * _D), lin2_W,
                          lin2_b.reshape(1, _D))
    return outp.reshape(_B, _NPAD)[:, :_NPER].reshape(-1)


# speedup vs baseline: 1.1411x; 1.1411x over previous
import functools

import numpy as np
import jax
import jax.numpy as jnp
from jax import lax
from jax.experimental import pallas as pl
from jax.experimental.pallas import tpu as pltpu
from jax.experimental.pallas import tpu_sc as plsc

_B = 20
_NPER = 500
_NPAD = 512
_EPER = 8000
_D = 128
_N = _B * _NPER
_NW = 32
_GROWS = 384
_GPAD = _NW * _GROWS
_K1, _K2, _K3 = 450, 405, 365

_f32 = jnp.float32
_i32 = jnp.int32
_SIGN = np.int32(-(2 ** 31))


def _sc_mesh():
    return plsc.VectorSubcoreMesh(core_axis_name="c", subcore_axis_name="s")


_GB = _B * _NPAD
_GW = _GB // _NW
_GC = 64


def _sc_fused(item_idx, cat_idx, item_table, cat_table, src, dst, zeros):
    @functools.partial(
        pl.kernel,
        mesh=_sc_mesh(),
        out_type=[jax.ShapeDtypeStruct((_GB, _D), _f32),
                  jax.ShapeDtypeStruct((_GB, _D), _f32),
                  jax.ShapeDtypeStruct((_B, 128 * _NPAD * 4), _f32)],
        scratch_types=[pltpu.VMEM((_GW,), _i32),
                       pltpu.VMEM((_GW,), _i32),
                       pltpu.VMEM((_GC, _D), _f32),
                       pltpu.VMEM((_GC, _D), _f32),
                       pltpu.VMEM((_GC, _D), _f32),
                       pltpu.VMEM((_GC, _D), _f32),
                       pltpu.VMEM((128 * _NPAD,), _f32),
                       pltpu.VMEM((_EPER,), _i32),
                       pltpu.VMEM((_EPER,), _i32),
                       pltpu.SemaphoreType.DMA,
                       pltpu.SemaphoreType.DMA,
                       pltpu.SemaphoreType.DMA,
                       pltpu.SemaphoreType.DMA,
                       pltpu.SemaphoreType.DMA,
                       pltpu.SemaphoreType.DMA,
                       pltpu.SemaphoreType.DMA,
                       pltpu.SemaphoreType.DMA],
        compiler_params=pltpu.CompilerParams(needs_layout_passes=False),
    )
    def fused_k(item_idx_h, cat_idx_h, item_t, cat_t, src_h, dst_h, zeros_h,
                item_o, cat_o, a_h,
                iidx, cidx, ib0, ib1, cb0, cb1, abuf, sbuf, dbuf,
                gi0, gi1, gc0, gc1, so0, so1, so2, so3):
        wid = lax.axis_index("s") * 2 + lax.axis_index("c")
        base = pl.multiple_of(wid * _GW, 64)
        pltpu.sync_copy(item_idx_h.at[pl.ds(base, _GW)], iidx)
        pltpu.sync_copy(cat_idx_h.at[pl.ds(base, _GW)], cidx)
        ibufs, cbufs = (ib0, ib1), (cb0, cb1)
        gis, gcs = (gi0, gi1), (gc0, gc1)
        sis, scs = (so0, so1), (so2, so3)
        ones = jnp.full((16,), 1.0, _f32)

        def fire_item(k):
            return pltpu.async_copy(
                item_t.at[iidx.at[pl.ds(k * _GC, _GC)]],
                ibufs[k % 2], gis[k % 2])

        def fire_cat(k):
            return pltpu.async_copy(
                cat_t.at[cidx.at[pl.ds(k * _GC, _GC)]],
                cbufs[k % 2], gcs[k % 2])

        def store_item(k):
            return pltpu.async_copy(
                ibufs[k % 2], item_o.at[pl.ds(base + k * _GC, _GC)],
                sis[k % 2])

        def store_cat(k):
            return pltpu.async_copy(
                cbufs[k % 2], cat_o.at[pl.ds(base + k * _GC, _GC)],
                scs[k % 2])

        def a_task(t):
            task = t * _NW + wid

            @pl.when(task < _B * 4)
            def _():
                g = task // 4
                ch = task - g * 4
                row0 = ch * 128
                node0 = g * _NPER
                e0 = pl.multiple_of(g * _EPER, 8)
                pltpu.sync_copy(zeros_h, abuf)
                pltpu.sync_copy(src_h.at[pl.ds(e0, _EPER)], sbuf)
                pltpu.sync_copy(dst_h.at[pl.ds(e0, _EPER)], dbuf)

                def body(e, carry):
                    for u in range(4):
                        off = pl.multiple_of(e * 64 + u * 16, 8)
                        s = sbuf[pl.ds(off, 16)] - node0
                        d = dbuf[pl.ds(off, 16)] - (node0 + row0)
                        m = (d >= 0) & (d < 128)
                        flat = jnp.where(m, d, 0) * _NPAD + s
                        plsc.addupdate_scatter(abuf, [flat], ones, mask=m)
                    return carry

                lax.fori_loop(0, _EPER // 64, body, 0)
                dst0 = pl.multiple_of(ch * (128 * _NPAD), 8)
                pltpu.sync_copy(abuf, a_h.at[g, pl.ds(dst0, 128 * _NPAD)])

        g_i0, g_c0 = fire_item(0), fire_cat(0)
        g_i1, g_c1 = fire_item(1), fire_cat(1)
        a_task(0)
        g_i0.wait()
        st_i0 = store_item(0)
        g_c0.wait()
        st_c0 = store_cat(0)
        a_task(1)
        g_i1.wait()
        st_i1 = store_item(1)
        g_c1.wait()
        st_c1 = store_cat(1)
        st_i0.wait()
        g_i2 = fire_item(2)
        st_c0.wait()
        g_c2 = fire_cat(2)
        a_task(2)
        g_i2.wait()
        st_i2 = store_item(2)
        g_c2.wait()
        st_c2 = store_cat(2)
        st_i1.wait()
        g_i3 = fire_item(3)
        st_c1.wait()
        g_c3 = fire_cat(3)
        g_i3.wait()
        st_i3 = store_item(3)
        g_c3.wait()
        st_c3 = store_cat(3)
        st_i2.wait()
        g_i4 = fire_item(4)
        st_c2.wait()
        g_c4 = fire_cat(4)
        g_i4.wait()
        st_i4 = store_item(4)
        g_c4.wait()
        st_c4 = store_cat(4)
        st_i3.wait()
        st_c3.wait()
        st_i4.wait()
        st_c4.wait()

    return fused_k(item_idx, cat_idx, item_table, cat_table, src, dst, zeros)


_bf16 = jnp.bfloat16


def _int_matmul_split(a16, h):
    h_hi = h.astype(_bf16)
    h_lo = (h - h_hi.astype(_f32)).astype(_bf16)
    return (jnp.dot(a16, h_hi, preferred_element_type=_f32)
            + jnp.dot(a16, h_lo, preferred_element_type=_f32))


_GPB = 4
_NG = _B // _GPB


def _conv1_call(A, hi, hc, W_rel, W_root, b, p):
    def body(a_ref, hi_ref, hc_ref, wr_ref, wo_ref, b_ref, p_ref,
             h1_ref, sc_ref):
        pv = p_ref[...]
        pn = jnp.sqrt(jnp.sum(pv * pv))
        valid = lax.broadcasted_iota(_i32, (_NPAD, 1), 0) < _NPER
        for j in range(_GPB):
            h0b = jnp.concatenate([hi_ref[j], hc_ref[j]], axis=1)
            hr = jnp.dot(h0b, wr_ref[...], preferred_element_type=_f32)
            h1 = _int_matmul_split(a_ref[j], hr)
            h1 = h1 + jnp.dot(h0b, wo_ref[...], preferred_element_type=_f32)
            h1 = jnp.maximum(h1 + b_ref[...], 0.0)
            raw = jnp.sum(h1 * pv, axis=1, keepdims=True) / pn
            sc_ref[j] = jnp.where(valid, raw, -jnp.inf)
            h1_ref[j] = h1

    return pl.pallas_call(
        body,
        grid=(_NG,),
        in_specs=[
            pl.BlockSpec((_GPB, _NPAD, _NPAD), lambda g: (g, 0, 0)),
            pl.BlockSpec((_GPB, _NPAD, _D), lambda g: (g, 0, 0)),
            pl.BlockSpec((_GPB, _NPAD, _D), lambda g: (g, 0, 0)),
            pl.BlockSpec((2 * _D, _D), lambda g: (0, 0)),
            pl.BlockSpec((2 * _D, _D), lambda g: (0, 0)),
            pl.BlockSpec((1, _D), lambda g: (0, 0)),
            pl.BlockSpec((1, _D), lambda g: (0, 0)),
        ],
        out_specs=[
            pl.BlockSpec((_GPB, _NPAD, _D), lambda g: (g, 0, 0)),
            pl.BlockSpec((_GPB, _NPAD, 1), lambda g: (g, 0, 0)),
        ],
        out_shape=[jax.ShapeDtypeStruct((_B, _NPAD, _D), _f32),
                   jax.ShapeDtypeStruct((_B, _NPAD, 1), _f32)],
    )(A, hi, hc, W_rel, W_root, b, p)


def _conv_next(A, h, keep, gate, W_rel, W_root, b, p, k_prev):
    inv_k = 1.0 / float(k_prev)

    def body(a_ref, h_ref, k_ref, g_ref, wr_ref, wo_ref, b_ref, p_ref,
             x_ref, h2_ref, sc_ref):
        pv = p_ref[...]
        pn = jnp.sqrt(jnp.sum(pv * pv))
        for j in range(_GPB):
            kp = k_ref[j]
            gt = g_ref[j]
            hp = h_ref[j] * gt * kp
            kept = kp > 0.0
            gmp = jnp.max(jnp.where(kept, hp, -jnp.inf), axis=0,
                          keepdims=True)
            gap = jnp.sum(hp, axis=0, keepdims=True) * inv_k
            x_ref[j] = jnp.concatenate([gmp, gap], axis=1)
            hr = jnp.dot(hp, wr_ref[...], preferred_element_type=_f32)
            h2 = _int_matmul_split(a_ref[j], hr)
            h2 = h2 + jnp.dot(hp, wo_ref[...], preferred_element_type=_f32)
            h2 = jnp.maximum(h2 + b_ref[...], 0.0)
            raw = jnp.sum(h2 * pv, axis=1, keepdims=True) / pn
            sc_ref[j] = jnp.where(kept, raw, -jnp.inf)
            h2_ref[j] = h2

    return pl.pallas_call(
        body,
        grid=(_NG,),
        in_specs=[
            pl.BlockSpec((_GPB, _NPAD, _NPAD), lambda g: (g, 0, 0)),
            pl.BlockSpec((_GPB, _NPAD, _D), lambda g: (g, 0, 0)),
            pl.BlockSpec((_GPB, _NPAD, 1), lambda g: (g, 0, 0)),
            pl.BlockSpec((_GPB, _NPAD, 1), lambda g: (g, 0, 0)),
            pl.BlockSpec((_D, _D), lambda g: (0, 0)),
            pl.BlockSpec((_D, _D), lambda g: (0, 0)),
            pl.BlockSpec((1, _D), lambda g: (0, 0)),
            pl.BlockSpec((1, _D), lambda g: (0, 0)),
        ],
        out_specs=[
            pl.BlockSpec((_GPB, 1, 2 * _D), lambda g: (g, 0, 0)),
            pl.BlockSpec((_GPB, _NPAD, _D), lambda g: (g, 0, 0)),
            pl.BlockSpec((_GPB, _NPAD, 1), lambda g: (g, 0, 0)),
        ],
        out_shape=[jax.ShapeDtypeStruct((_B, 1, 2 * _D), _f32),
                   jax.ShapeDtypeStruct((_B, _NPAD, _D), _f32),
                   jax.ShapeDtypeStruct((_B, _NPAD, 1), _f32)],
    )(A, h, keep, gate, W_rel, W_root, b, p)


def _pool_select(score2d, k):
    def body(s_ref, keep_ref, gate_ref):
        s = s_ref[...]
        bits = lax.bitcast_convert_type(s, _i32)
        key = jnp.where(bits >= 0, bits, bits ^ np.int32(0x7FFFFFFF))
        pref = jnp.zeros((_B, 1), _i32)
        for bit in range(31, -1, -1):
            m = _SIGN if bit == 31 else np.int32(1 << bit)
            cand = pref | m
            t = cand ^ _SIGN
            c = jnp.sum((key >= t).astype(_i32), axis=1, keepdims=True)
            pref = jnp.where(c >= k, cand, pref)
        kth = pref ^ _SIGN
        gtm = key > kth
        eq = (key >= kth) & (~gtm)
        n_gt = jnp.sum(gtm.astype(_i32), axis=1, keepdims=True)
        run = eq.astype(_i32)
        for sh in (1, 2, 4, 8, 16, 32, 64, 128, 256):
            shifted = jnp.concatenate(
                [jnp.zeros((_B, sh), _i32), run[:, : _NPAD - sh]], axis=1)
            run = run + shifted
        keep = gtm | (eq & (run <= (k - n_gt)))
        keep_ref[...] = keep.astype(_f32)
        gate_ref[...] = jnp.tanh(jnp.maximum(s, -100.0))

    return pl.pallas_call(
        body,
        out_shape=[jax.ShapeDtypeStruct((_B, _NPAD), _f32),
                   jax.ShapeDtypeStruct((_B, _NPAD), _f32)],
    )(score2d)


def _final_call(h3, keep, gate, x1, x2, emb_item, l1W, l1b, l2W, l2b):
    inv_k = 1.0 / float(_K3)

    def body(h_ref, k_ref, g_ref, x1_ref, x2_ref, e_ref,
             w1_ref, b1_ref, w2_ref, b2_ref, o_ref):
        for j in range(_GPB):
            kp = k_ref[j]
            gt = g_ref[j]
            hp = h_ref[j] * gt * kp
            kept = kp > 0.0
            gmp = jnp.max(jnp.where(kept, hp, -jnp.inf), axis=0,
                          keepdims=True)
            gap = jnp.sum(hp, axis=0, keepdims=True) * inv_k
            gv = x1_ref[j] + x2_ref[j] + jnp.concatenate([gmp, gap], axis=1)
            gv = jnp.dot(gv, w1_ref[...], preferred_element_type=_f32)
            gv = jnp.maximum(gv + b1_ref[...], 0.0)
            gv = jnp.dot(gv, w2_ref[...], preferred_element_type=_f32)
            gv = jnp.maximum(gv + b2_ref[...], 0.0)
            logits = jnp.sum(e_ref[j] * gv, axis=1, keepdims=True)
            o_ref[j] = 1.0 / (1.0 + jnp.exp(-logits))

    return pl.pallas_call(
        body,
        grid=(_NG,),
        in_specs=[
            pl.BlockSpec((_GPB, _NPAD, _D), lambda g: (g, 0, 0)),
            pl.BlockSpec((_GPB, _NPAD, 1), lambda g: (g, 0, 0)),
            pl.BlockSpec((_GPB, _NPAD, 1), lambda g: (g, 0, 0)),
            pl.BlockSpec((_GPB, 1, 2 * _D), lambda g: (g, 0, 0)),
            pl.BlockSpec((_GPB, 1, 2 * _D), lambda g: (g, 0, 0)),
            pl.BlockSpec((_GPB, _NPAD, _D), lambda g: (g, 0, 0)),
            pl.BlockSpec((2 * _D, 2 * _D), lambda g: (0, 0)),
            pl.BlockSpec((1, 2 * _D), lambda g: (0, 0)),
            pl.BlockSpec((2 * _D, _D), lambda g: (0, 0)),
            pl.BlockSpec((1, _D), lambda g: (0, 0)),
        ],
        out_specs=[pl.BlockSpec((_GPB, _NPAD, 1), lambda g: (g, 0, 0))],
        out_shape=[jax.ShapeDtypeStruct((_B, _NPAD, 1), _f32)],
    )(h3, keep, gate, x1, x2, emb_item, l1W, l1b, l2W, l2b)


def kernel(x, edge_index, batch, item_table, cat_table, W1_rel, W1_root, b1,
           p1, W2_rel, W2_root, b2, p2, W3_rel, W3_root, b3, p3, lin1_W,
           lin1_b, lin2_W, lin2_b):
    idx_pad = jnp.zeros((_B, _NPAD - _NPER), _i32)
    item_idx = jnp.concatenate(
        [x[:, 0, 0].astype(_i32).reshape(_B, _NPER), idx_pad],
        axis=1).reshape(-1)
    cat_idx = jnp.concatenate(
        [x[:, 0, 1].astype(_i32).reshape(_B, _NPER), idx_pad],
        axis=1).reshape(-1)
    src = edge_index[0].astype(_i32)
    dst = edge_index[1].astype(_i32)
    emb_item_f, emb_cat_f, A_flat = _sc_fused(
        item_idx, cat_idx, item_table, cat_table, src, dst,
        jnp.zeros((128 * _NPAD,), _f32))
    emb_item = emb_item_f.reshape(_B, _NPAD, _D)
    emb_cat = emb_cat_f.reshape(_B, _NPAD, _D)
    A = A_flat.reshape(_B, _NPAD, _NPAD).astype(_bf16)

    h1, s1 = _conv1_call(A, emb_item, emb_cat, W1_rel, W1_root,
                         b1.reshape(1, _D), p1.reshape(1, _D))
    k1, g1 = _pool_select(s1.reshape(_B, _NPAD), _K1)
    keep1 = k1.reshape(_B, _NPAD, 1)
    gate1 = g1.reshape(_B, _NPAD, 1)

    x1, h2, s2 = _conv_next(A, h1, keep1, gate1, W2_rel, W2_root,
                            b2.reshape(1, _D), p2.reshape(1, _D), _K1)
    k2, g2 = _pool_select(s2.reshape(_B, _NPAD), _K2)
    keep2 = k2.reshape(_B, _NPAD, 1)
    gate2 = g2.reshape(_B, _NPAD, 1)

    x2, h3, s3 = _conv_next(A, h2, keep2, gate2, W3_rel, W3_root,
                            b3.reshape(1, _D), p3.reshape(1, _D), _K2)
    k3, g3 = _pool_select(s3.reshape(_B, _NPAD), _K3)
    keep3 = k3.reshape(_B, _NPAD, 1)
    gate3 = g3.reshape(_B, _NPAD, 1)

    (outp,) = _final_call(h3, keep3, gate3, x1, x2, emb_item,
                          lin1_W,
                          lin1_b.reshape(1, 2 * _D), lin2_W,
                          lin2_b.reshape(1, _D))
    return outp.reshape(_B, _NPAD)[:, :_NPER].reshape(-1)


# speedup vs baseline: 1.1439x; 1.0025x over previous
import functools

import numpy as np
import jax
import jax.numpy as jnp
from jax import lax
from jax.experimental import pallas as pl
from jax.experimental.pallas import tpu as pltpu
from jax.experimental.pallas import tpu_sc as plsc

_B = 20
_NPER = 500
_NPAD = 512
_EPER = 8000
_D = 128
_N = _B * _NPER
_NW = 32
_GROWS = 384
_GPAD = _NW * _GROWS
_K1, _K2, _K3 = 450, 405, 365

_f32 = jnp.float32
_i32 = jnp.int32
_SIGN = np.int32(-(2 ** 31))


def _sc_mesh():
    return plsc.VectorSubcoreMesh(core_axis_name="c", subcore_axis_name="s")


_GB = _B * _NPAD
_GW = _GB // _NW
_GC = 64


def _sc_fused(item_idx, cat_idx, item_table, cat_table, src, dst, zeros):
    @functools.partial(
        pl.kernel,
        mesh=_sc_mesh(),
        out_type=[jax.ShapeDtypeStruct((_GB, _D), _f32),
                  jax.ShapeDtypeStruct((_GB, _D), _f32),
                  jax.ShapeDtypeStruct((_B, 128 * _NPAD * 4), _f32)],
        scratch_types=[pltpu.VMEM((_GW,), _i32),
                       pltpu.VMEM((_GW,), _i32),
                       pltpu.VMEM((_GC, _D), _f32),
                       pltpu.VMEM((_GC, _D), _f32),
                       pltpu.VMEM((_GC, _D), _f32),
                       pltpu.VMEM((_GC, _D), _f32),
                       pltpu.VMEM((128 * _NPAD,), _f32),
                       pltpu.VMEM((_EPER,), _i32),
                       pltpu.VMEM((_EPER,), _i32),
                       pltpu.SemaphoreType.DMA,
                       pltpu.SemaphoreType.DMA,
                       pltpu.SemaphoreType.DMA,
                       pltpu.SemaphoreType.DMA,
                       pltpu.SemaphoreType.DMA,
                       pltpu.SemaphoreType.DMA,
                       pltpu.SemaphoreType.DMA,
                       pltpu.SemaphoreType.DMA],
        compiler_params=pltpu.CompilerParams(needs_layout_passes=False),
    )
    def fused_k(item_idx_h, cat_idx_h, item_t, cat_t, src_h, dst_h, zeros_h,
                item_o, cat_o, a_h,
                iidx, cidx, ib0, ib1, cb0, cb1, abuf, sbuf, dbuf,
                gi0, gi1, gc0, gc1, so0, so1, so2, so3):
        wid = lax.axis_index("s") * 2 + lax.axis_index("c")
        base = pl.multiple_of(wid * _GW, 64)
        pltpu.sync_copy(item_idx_h.at[pl.ds(base, _GW)], iidx)
        pltpu.sync_copy(cat_idx_h.at[pl.ds(base, _GW)], cidx)
        ibufs, cbufs = (ib0, ib1), (cb0, cb1)
        gis, gcs = (gi0, gi1), (gc0, gc1)
        sis, scs = (so0, so1), (so2, so3)
        ones = jnp.full((16,), 1.0, _f32)

        def fire_item(k):
            return pltpu.async_copy(
                item_t.at[iidx.at[pl.ds(k * _GC, _GC)]],
                ibufs[k % 2], gis[k % 2])

        def fire_cat(k):
            return pltpu.async_copy(
                cat_t.at[cidx.at[pl.ds(k * _GC, _GC)]],
                cbufs[k % 2], gcs[k % 2])

        def store_item(k):
            return pltpu.async_copy(
                ibufs[k % 2], item_o.at[pl.ds(base + k * _GC, _GC)],
                sis[k % 2])

        def store_cat(k):
            return pltpu.async_copy(
                cbufs[k % 2], cat_o.at[pl.ds(base + k * _GC, _GC)],
                scs[k % 2])

        def a_task(t):
            task = t * _NW + wid

            @pl.when(task < _B * 4)
            def _():
                g = task // 4
                ch = task - g * 4
                row0 = ch * 128
                node0 = g * _NPER
                e0 = pl.multiple_of(g * _EPER, 8)
                pltpu.sync_copy(zeros_h, abuf)
                pltpu.sync_copy(src_h.at[pl.ds(e0, _EPER)], sbuf)
                pltpu.sync_copy(dst_h.at[pl.ds(e0, _EPER)], dbuf)

                def body(e, carry):
                    for u in range(4):
                        off = pl.multiple_of(e * 64 + u * 16, 8)
                        s = sbuf[pl.ds(off, 16)] - node0
                        d = dbuf[pl.ds(off, 16)] - (node0 + row0)
                        m = (d >= 0) & (d < 128)
                        flat = jnp.where(m, d, 0) * _NPAD + s
                        plsc.addupdate_scatter(abuf, [flat], ones, mask=m)
                    return carry

                lax.fori_loop(0, _EPER // 64, body, 0)
                dst0 = pl.multiple_of(ch * (128 * _NPAD), 8)
                pltpu.sync_copy(abuf, a_h.at[g, pl.ds(dst0, 128 * _NPAD)])

        g_i0, g_c0 = fire_item(0), fire_cat(0)
        g_i1, g_c1 = fire_item(1), fire_cat(1)
        a_task(0)
        g_i0.wait()
        st_i0 = store_item(0)
        g_c0.wait()
        st_c0 = store_cat(0)
        a_task(1)
        g_i1.wait()
        st_i1 = store_item(1)
        g_c1.wait()
        st_c1 = store_cat(1)
        st_i0.wait()
        g_i2 = fire_item(2)
        st_c0.wait()
        g_c2 = fire_cat(2)
        a_task(2)
        g_i2.wait()
        st_i2 = store_item(2)
        g_c2.wait()
        st_c2 = store_cat(2)
        st_i1.wait()
        g_i3 = fire_item(3)
        st_c1.wait()
        g_c3 = fire_cat(3)
        g_i3.wait()
        st_i3 = store_item(3)
        g_c3.wait()
        st_c3 = store_cat(3)
        st_i2.wait()
        g_i4 = fire_item(4)
        st_c2.wait()
        g_c4 = fire_cat(4)
        g_i4.wait()
        st_i4 = store_item(4)
        g_c4.wait()
        st_c4 = store_cat(4)
        st_i3.wait()
        st_c3.wait()
        st_i4.wait()
        st_c4.wait()

    return fused_k(item_idx, cat_idx, item_table, cat_table, src, dst, zeros)


_bf16 = jnp.bfloat16


def _int_matmul_split(a16, h):
    h_hi = h.astype(_bf16)
    h_lo = (h - h_hi.astype(_f32)).astype(_bf16)
    return (jnp.dot(a16, h_hi, preferred_element_type=_f32)
            + jnp.dot(a16, h_lo, preferred_element_type=_f32))


_GPB = 5
_NG = _B // _GPB


def _conv1_call(A, hi, hc, W_rel, W_root, b, p):
    def body(a_ref, hi_ref, hc_ref, wr_ref, wo_ref, b_ref, p_ref,
             h1_ref, sc_ref):
        pv = p_ref[...]
        pn = jnp.sqrt(jnp.sum(pv * pv))
        valid = lax.broadcasted_iota(_i32, (_NPAD, 1), 0) < _NPER
        for j in range(_GPB):
            h0b = jnp.concatenate([hi_ref[j], hc_ref[j]], axis=1)
            hr = jnp.dot(h0b, wr_ref[...], preferred_element_type=_f32)
            h1 = _int_matmul_split(a_ref[j], hr)
            h1 = h1 + jnp.dot(h0b, wo_ref[...], preferred_element_type=_f32)
            h1 = jnp.maximum(h1 + b_ref[...], 0.0)
            raw = jnp.sum(h1 * pv, axis=1, keepdims=True) / pn
            sc_ref[j] = jnp.where(valid, raw, -jnp.inf)
            h1_ref[j] = h1

    return pl.pallas_call(
        body,
        grid=(_NG,),
        in_specs=[
            pl.BlockSpec((_GPB, _NPAD, _NPAD), lambda g: (g, 0, 0)),
            pl.BlockSpec((_GPB, _NPAD, _D), lambda g: (g, 0, 0)),
            pl.BlockSpec((_GPB, _NPAD, _D), lambda g: (g, 0, 0)),
            pl.BlockSpec((2 * _D, _D), lambda g: (0, 0)),
            pl.BlockSpec((2 * _D, _D), lambda g: (0, 0)),
            pl.BlockSpec((1, _D), lambda g: (0, 0)),
            pl.BlockSpec((1, _D), lambda g: (0, 0)),
        ],
        out_specs=[
            pl.BlockSpec((_GPB, _NPAD, _D), lambda g: (g, 0, 0)),
            pl.BlockSpec((_GPB, _NPAD, 1), lambda g: (g, 0, 0)),
        ],
        out_shape=[jax.ShapeDtypeStruct((_B, _NPAD, _D), _f32),
                   jax.ShapeDtypeStruct((_B, _NPAD, 1), _f32)],
    )(A, hi, hc, W_rel, W_root, b, p)


def _conv_next(A, h, keep, gate, W_rel, W_root, b, p, k_prev):
    inv_k = 1.0 / float(k_prev)

    def body(a_ref, h_ref, k_ref, g_ref, wr_ref, wo_ref, b_ref, p_ref,
             x_ref, h2_ref, sc_ref):
        pv = p_ref[...]
        pn = jnp.sqrt(jnp.sum(pv * pv))
        for j in range(_GPB):
            kp = k_ref[j]
            gt = g_ref[j]
            hp = h_ref[j] * gt * kp
            kept = kp > 0.0
            gmp = jnp.max(jnp.where(kept, hp, -jnp.inf), axis=0,
                          keepdims=True)
            gap = jnp.sum(hp, axis=0, keepdims=True) * inv_k
            x_ref[j] = jnp.concatenate([gmp, gap], axis=1)
            hr = jnp.dot(hp, wr_ref[...], preferred_element_type=_f32)
            h2 = _int_matmul_split(a_ref[j], hr)
            h2 = h2 + jnp.dot(hp, wo_ref[...], preferred_element_type=_f32)
            h2 = jnp.maximum(h2 + b_ref[...], 0.0)
            raw = jnp.sum(h2 * pv, axis=1, keepdims=True) / pn
            sc_ref[j] = jnp.where(kept, raw, -jnp.inf)
            h2_ref[j] = h2

    return pl.pallas_call(
        body,
        grid=(_NG,),
        in_specs=[
            pl.BlockSpec((_GPB, _NPAD, _NPAD), lambda g: (g, 0, 0)),
            pl.BlockSpec((_GPB, _NPAD, _D), lambda g: (g, 0, 0)),
            pl.BlockSpec((_GPB, _NPAD, 1), lambda g: (g, 0, 0)),
            pl.BlockSpec((_GPB, _NPAD, 1), lambda g: (g, 0, 0)),
            pl.BlockSpec((_D, _D), lambda g: (0, 0)),
            pl.BlockSpec((_D, _D), lambda g: (0, 0)),
            pl.BlockSpec((1, _D), lambda g: (0, 0)),
            pl.BlockSpec((1, _D), lambda g: (0, 0)),
        ],
        out_specs=[
            pl.BlockSpec((_GPB, 1, 2 * _D), lambda g: (g, 0, 0)),
            pl.BlockSpec((_GPB, _NPAD, _D), lambda g: (g, 0, 0)),
            pl.BlockSpec((_GPB, _NPAD, 1), lambda g: (g, 0, 0)),
        ],
        out_shape=[jax.ShapeDtypeStruct((_B, 1, 2 * _D), _f32),
                   jax.ShapeDtypeStruct((_B, _NPAD, _D), _f32),
                   jax.ShapeDtypeStruct((_B, _NPAD, 1), _f32)],
    )(A, h, keep, gate, W_rel, W_root, b, p)


def _pool_select(score2d, k):
    def body(s_ref, keep_ref, gate_ref):
        s = s_ref[...]
        bits = lax.bitcast_convert_type(s, _i32)
        key = jnp.where(bits >= 0, bits, bits ^ np.int32(0x7FFFFFFF))
        pref = jnp.zeros((_B, 1), _i32)
        for bit in range(31, -1, -1):
            m = _SIGN if bit == 31 else np.int32(1 << bit)
            cand = pref | m
            t = cand ^ _SIGN
            c = jnp.sum((key >= t).astype(_i32), axis=1, keepdims=True)
            pref = jnp.where(c >= k, cand, pref)
        kth = pref ^ _SIGN
        gtm = key > kth
        eq = (key >= kth) & (~gtm)
        n_gt = jnp.sum(gtm.astype(_i32), axis=1, keepdims=True)
        run = eq.astype(_i32)
        for sh in (1, 2, 4, 8, 16, 32, 64, 128, 256):
            shifted = jnp.concatenate(
                [jnp.zeros((_B, sh), _i32), run[:, : _NPAD - sh]], axis=1)
            run = run + shifted
        keep = gtm | (eq & (run <= (k - n_gt)))
        keep_ref[...] = keep.astype(_f32)
        gate_ref[...] = jnp.tanh(jnp.maximum(s, -100.0))

    return pl.pallas_call(
        body,
        out_shape=[jax.ShapeDtypeStruct((_B, _NPAD), _f32),
                   jax.ShapeDtypeStruct((_B, _NPAD), _f32)],
    )(score2d)


def _final_call(h3, keep, gate, x1, x2, emb_item, l1W, l1b, l2W, l2b):
    inv_k = 1.0 / float(_K3)

    def body(h_ref, k_ref, g_ref, x1_ref, x2_ref, e_ref,
             w1_ref, b1_ref, w2_ref, b2_ref, o_ref):
        for j in range(_GPB):
            kp = k_ref[j]
            gt = g_ref[j]
            hp = h_ref[j] * gt * kp
            kept = kp > 0.0
            gmp = jnp.max(jnp.where(kept, hp, -jnp.inf), axis=0,
                          keepdims=True)
            gap = jnp.sum(hp, axis=0, keepdims=True) * inv_k
            gv = x1_ref[j] + x2_ref[j] + jnp.concatenate([gmp, gap], axis=1)
            gv = jnp.dot(gv, w1_ref[...], preferred_element_type=_f32)
            gv = jnp.maximum(gv + b1_ref[...], 0.0)
            gv = jnp.dot(gv, w2_ref[...], preferred_element_type=_f32)
            gv = jnp.maximum(gv + b2_ref[...], 0.0)
            logits = jnp.sum(e_ref[j] * gv, axis=1, keepdims=True)
            o_ref[j] = 1.0 / (1.0 + jnp.exp(-logits))

    return pl.pallas_call(
        body,
        grid=(_NG,),
        in_specs=[
            pl.BlockSpec((_GPB, _NPAD, _D), lambda g: (g, 0, 0)),
            pl.BlockSpec((_GPB, _NPAD, 1), lambda g: (g, 0, 0)),
            pl.BlockSpec((_GPB, _NPAD, 1), lambda g: (g, 0, 0)),
            pl.BlockSpec((_GPB, 1, 2 * _D), lambda g: (g, 0, 0)),
            pl.BlockSpec((_GPB, 1, 2 * _D), lambda g: (g, 0, 0)),
            pl.BlockSpec((_GPB, _NPAD, _D), lambda g: (g, 0, 0)),
            pl.BlockSpec((2 * _D, 2 * _D), lambda g: (0, 0)),
            pl.BlockSpec((1, 2 * _D), lambda g: (0, 0)),
            pl.BlockSpec((2 * _D, _D), lambda g: (0, 0)),
            pl.BlockSpec((1, _D), lambda g: (0, 0)),
        ],
        out_specs=[pl.BlockSpec((_GPB, _NPAD, 1), lambda g: (g, 0, 0))],
        out_shape=[jax.ShapeDtypeStruct((_B, _NPAD, 1), _f32)],
    )(h3, keep, gate, x1, x2, emb_item, l1W, l1b, l2W, l2b)


def kernel(x, edge_index, batch, item_table, cat_table, W1_rel, W1_root, b1,
           p1, W2_rel, W2_root, b2, p2, W3_rel, W3_root, b3, p3, lin1_W,
           lin1_b, lin2_W, lin2_b):
    idx_pad = jnp.zeros((_B, _NPAD - _NPER), _i32)
    item_idx = jnp.concatenate(
        [x[:, 0, 0].astype(_i32).reshape(_B, _NPER), idx_pad],
        axis=1).reshape(-1)
    cat_idx = jnp.concatenate(
        [x[:, 0, 1].astype(_i32).reshape(_B, _NPER), idx_pad],
        axis=1).reshape(-1)
    src = edge_index[0].astype(_i32)
    dst = edge_index[1].astype(_i32)
    emb_item_f, emb_cat_f, A_flat = _sc_fused(
        item_idx, cat_idx, item_table, cat_table, src, dst,
        jnp.zeros((128 * _NPAD,), _f32))
    emb_item = emb_item_f.reshape(_B, _NPAD, _D)
    emb_cat = emb_cat_f.reshape(_B, _NPAD, _D)
    A = A_flat.reshape(_B, _NPAD, _NPAD).astype(_bf16)

    h1, s1 = _conv1_call(A, emb_item, emb_cat, W1_rel, W1_root,
                         b1.reshape(1, _D), p1.reshape(1, _D))
    k1, g1 = _pool_select(s1.reshape(_B, _NPAD), _K1)
    keep1 = k1.reshape(_B, _NPAD, 1)
    gate1 = g1.reshape(_B, _NPAD, 1)

    x1, h2, s2 = _conv_next(A, h1, keep1, gate1, W2_rel, W2_root,
                            b2.reshape(1, _D), p2.reshape(1, _D), _K1)
    k2, g2 = _pool_select(s2.reshape(_B, _NPAD), _K2)
    keep2 = k2.reshape(_B, _NPAD, 1)
    gate2 = g2.reshape(_B, _NPAD, 1)

    x2, h3, s3 = _conv_next(A, h2, keep2, gate2, W3_rel, W3_root,
                            b3.reshape(1, _D), p3.reshape(1, _D), _K2)
    k3, g3 = _pool_select(s3.reshape(_B, _NPAD), _K3)
    keep3 = k3.reshape(_B, _NPAD, 1)
    gate3 = g3.reshape(_B, _NPAD, 1)

    (outp,) = _final_call(h3, keep3, gate3, x1, x2, emb_item,
                          lin1_W,
                          lin1_b.reshape(1, 2 * _D), lin2_W,
                          lin2_b.reshape(1, _D))
    return outp.reshape(_B, _NPAD)[:, :_NPER].reshape(-1)
